# baseline (speedup 1.0000x reference)
import functools

import jax
import jax.numpy as jnp
from jax import lax
from jax.experimental import pallas as pl
from jax.experimental.pallas import tpu as pltpu
from jax.experimental.pallas import tpu_sc as plsc

_L = 128
_R = 4


@functools.lru_cache(maxsize=None)
def _make_kernel(n_idx, d):
    info = plsc.get_sparse_core_info()
    nc, ns = info.num_cores, info.num_subcores
    nw = nc * ns
    rows_total = n_idx // _L
    rows_per_w = rows_total // nw
    n_chunks = rows_per_w // _R
    mesh = plsc.VectorSubcoreMesh(core_axis_name="c", subcore_axis_name="s")

    @functools.partial(
        pl.kernel,
        mesh=mesh,
        compiler_params=pltpu.CompilerParams(use_tc_tiling_on_sc=False),
        out_type=[
            jax.ShapeDtypeStruct((n_idx, d), jnp.float32),
            jax.ShapeDtypeStruct((n_idx, d), jnp.float32),
        ],
        scratch_types=[
            pltpu.VMEM((_R, _L), jnp.int32),
            pltpu.VMEM((_R * _L, d), jnp.float32),
            pltpu.VMEM((_R * _L, d), jnp.float32),
            pltpu.SemaphoreType.DMA,
        ],
    )
    def k(idx_hbm, rtab, itab, out_r, out_i, idx_v, rrows, irows, sem):
        wid = lax.axis_index("s") * nc + lax.axis_index("c")
        row0 = wid * rows_per_w

        def chunk(g, carry):
            base_row = row0 + g * _R
            pltpu.sync_copy(idx_hbm.at[pl.ds(base_row, _R)], idx_v)
            cps = []
            for j in range(_R):
                cps.append(pltpu.async_copy(
                    rtab.at[idx_v.at[j]], rrows.at[pl.ds(j * _L, _L)], sem))
                cps.append(pltpu.async_copy(
                    itab.at[idx_v.at[j]], irows.at[pl.ds(j * _L, _L)], sem))
            for cp in cps:
                cp.wait()
            ob = base_row * _L
            pltpu.sync_copy(rrows, out_r.at[pl.ds(ob, _R * _L)])
            pltpu.sync_copy(irows, out_i.at[pl.ds(ob, _R * _L)])
            return carry

        lax.fori_loop(0, n_chunks, chunk, 0)

    return k


def kernel(x, real_embedding, imag_embedding):
    b, h = x.shape
    n = b * h
    d = real_embedding.shape[1]
    idx = x.reshape(n // _L, _L).astype(jnp.int32)
    out_r, out_i = _make_kernel(n, d)(idx, real_embedding, imag_embedding)
    return out_r.reshape(b, h, d), out_i.reshape(b, h, d)


# speedup vs baseline: 1.1142x; 1.1142x over previous
import functools

import jax
import jax.numpy as jnp
from jax import lax
from jax.experimental import pallas as pl
from jax.experimental.pallas import tpu as pltpu
from jax.experimental.pallas import tpu_sc as plsc

_L = 128
_RC = 2


@functools.lru_cache(maxsize=None)
def _make_kernel(n_idx, dd):
    info = plsc.get_sparse_core_info()
    nc, ns = info.num_cores, info.num_subcores
    nw = nc * ns
    rows_per_w = (n_idx // _L) // nw
    n_chunks = rows_per_w // _RC
    cpw = _RC * _L
    mesh = plsc.VectorSubcoreMesh(core_axis_name="c", subcore_axis_name="s")

    @functools.partial(
        pl.kernel,
        mesh=mesh,
        compiler_params=pltpu.CompilerParams(use_tc_tiling_on_sc=False),
        out_type=jax.ShapeDtypeStruct((n_idx, dd), jnp.float32),
        scratch_types=[
            pltpu.VMEM((rows_per_w, _L), jnp.int32),
            pltpu.VMEM((2, cpw, dd), jnp.float32),
            pltpu.SemaphoreType.DMA,
            pltpu.SemaphoreType.DMA,
            pltpu.SemaphoreType.DMA,
            pltpu.SemaphoreType.DMA,
        ],
    )
    def k(idx_hbm, tab, out, idx_v, st, gsa, gsb, wsa, wsb):
        wid = lax.axis_index("s") * nc + lax.axis_index("c")
        row0 = wid * rows_per_w
        pltpu.sync_copy(idx_hbm.at[pl.ds(row0, rows_per_w)], idx_v)

        def g_parts(t, b, sem):
            return [
                (tab.at[idx_v.at[t * _RC + j]],
                 st.at[b, pl.ds(j * _L, _L)], sem)
                for j in range(_RC)
            ]

        def w_parts(t, b, sem):
            ob = (row0 + t * _RC) * _L
            return [(st.at[b], out.at[pl.ds(ob, cpw)], sem)]

        def issue(parts):
            for src, dst, sem in parts:
                pltpu.async_copy(src, dst, sem)

        def drain(parts):
            for src, dst, sem in parts:
                pltpu.make_async_copy(src, dst, sem).wait()

        last = n_chunks - 1
        issue(g_parts(0, 0, gsa))

        def body(u, carry):
            t0 = 2 * u
            t1 = 2 * u + 1
            drain(g_parts(t0, 0, gsa))
            issue(w_parts(t0, 0, wsa))
            issue(g_parts(t1, 1, gsb))
            drain(g_parts(t1, 1, gsb))
            issue(w_parts(t1, 1, wsb))
            drain(w_parts(t0, 0, wsa))
            t2 = jnp.minimum(t0 + 2, last)
            issue(g_parts(t2, 0, gsa))
            drain(w_parts(t1, 1, wsb))
            return carry

        lax.fori_loop(0, n_chunks // 2, body, 0)
        drain(g_parts(last, 0, gsa))

    return k


def kernel(x, real_embedding, imag_embedding):
    b, h = x.shape
    d = real_embedding.shape[1]
    n = b * h
    idx = x.reshape(n // _L, _L).astype(jnp.int32)
    tab = jnp.concatenate([real_embedding, imag_embedding], axis=1)
    big = _make_kernel(n, 2 * d)(idx, tab)
    return (big[:, :d].reshape(b, h, d), big[:, d:].reshape(b, h, d))


# speedup vs baseline: 1.1167x; 1.0023x over previous
import functools

import jax
import jax.numpy as jnp
from jax import lax
from jax.experimental import pallas as pl
from jax.experimental.pallas import tpu as pltpu
from jax.experimental.pallas import tpu_sc as plsc

_L = 128
_RC = 2


@functools.lru_cache(maxsize=None)
def _make_kernel(n_idx, dd):
    info = plsc.get_sparse_core_info()
    nc, ns = info.num_cores, info.num_subcores
    nw = nc * ns
    rows_per_w = (n_idx // _L) // nw
    n_chunks = rows_per_w // _RC
    cpw = _RC * _L
    mesh = plsc.VectorSubcoreMesh(core_axis_name="c", subcore_axis_name="s")

    @functools.partial(
        pl.kernel,
        mesh=mesh,
        out_type=jax.ShapeDtypeStruct((n_idx, dd), jnp.float32),
        scratch_types=[
            pltpu.VMEM((rows_per_w, _L), jnp.int32),
            pltpu.VMEM((2, cpw, dd), jnp.float32),
            pltpu.SemaphoreType.DMA,
            pltpu.SemaphoreType.DMA,
            pltpu.SemaphoreType.DMA,
            pltpu.SemaphoreType.DMA,
        ],
    )
    def k(idx_hbm, tab, out, idx_v, st, gsa, gsb, wsa, wsb):
        wid = lax.axis_index("s") * nc + lax.axis_index("c")
        row0 = wid * rows_per_w
        pltpu.sync_copy(idx_hbm.at[pl.ds(row0, rows_per_w)], idx_v)

        def g_parts(t, b, sem):
            return [
                (tab.at[idx_v.at[t * _RC + j]],
                 st.at[b, pl.ds(j * _L, _L)], sem)
                for j in range(_RC)
            ]

        def w_parts(t, b, sem):
            ob = (row0 + t * _RC) * _L
            return [(st.at[b], out.at[pl.ds(ob, cpw)], sem)]

        def issue(parts):
            for src, dst, sem in parts:
                pltpu.async_copy(src, dst, sem)

        def drain(parts):
            for src, dst, sem in parts:
                pltpu.make_async_copy(src, dst, sem).wait()

        last = n_chunks - 1
        issue(g_parts(0, 0, gsa))

        def body(u, carry):
            t0 = 2 * u
            t1 = 2 * u + 1
            drain(g_parts(t0, 0, gsa))
            issue(w_parts(t0, 0, wsa))
            issue(g_parts(t1, 1, gsb))
            drain(g_parts(t1, 1, gsb))
            issue(w_parts(t1, 1, wsb))
            drain(w_parts(t0, 0, wsa))
            t2 = jnp.minimum(t0 + 2, last)
            issue(g_parts(t2, 0, gsa))
            drain(w_parts(t1, 1, wsb))
            return carry

        lax.fori_loop(0, n_chunks // 2, body, 0)
        drain(g_parts(last, 0, gsa))

    return k


def kernel(x, real_embedding, imag_embedding):
    b, h = x.shape
    d = real_embedding.shape[1]
    n = b * h
    idx = x.reshape(n // _L, _L).astype(jnp.int32)
    tab = jnp.concatenate([real_embedding, imag_embedding], axis=1)
    big = _make_kernel(n, 2 * d)(idx, tab)
    return (big[:, :d].reshape(b, h, d), big[:, d:].reshape(b, h, d))
